# Initial kernel scaffold; baseline (speedup 1.0000x reference)
import jax
import jax.numpy as jnp
from jax.experimental import pallas as pl


def kernel(clause_emb, Wq_w, Wq_b, Wk_w, Wk_b, keep_mask):
    raise NotImplementedError("write your pallas kernel here")


# speedup vs baseline: 14.1154x; 14.1154x over previous
import math

import jax
import jax.numpy as jnp
from jax.experimental import pallas as pl
from jax.experimental.pallas import tpu as pltpu

_N = 2048
_H = 2048
_TOPK = 8
_TRI = _N * (_N - 1) // 2
_SCALE = 1.0 / math.sqrt(_H)
_BIG = 2**30
_NINF = float("-inf")

_BM = 512
_BN = 512
_BK = 512

_DIMS = (((1,), (1,)), ((), ()))


def _mm3(ah, al, bh, bl):
    return (
        jax.lax.dot_general(ah, bh, _DIMS, preferred_element_type=jnp.float32)
        + jax.lax.dot_general(ah, bl, _DIMS, preferred_element_type=jnp.float32)
        + jax.lax.dot_general(al, bh, _DIMS, preferred_element_type=jnp.float32)
    )


def _split(x):
    hi = x.astype(jnp.bfloat16)
    lo = (x - hi.astype(jnp.float32)).astype(jnp.bfloat16)
    return hi, lo


def _qk_kernel(xh, xl, wqh, wql, wkh, wkl, bq, bk, q_out, k_out):
    kk = pl.program_id(2)

    @pl.when(kk == 0)
    def _():
        q_out[...] = jnp.zeros_like(q_out)
        k_out[...] = jnp.zeros_like(k_out)

    xh_, xl_ = xh[...], xl[...]
    q_out[...] += _mm3(xh_, xl_, wqh[...], wql[...])
    k_out[...] += _mm3(xh_, xl_, wkh[...], wkl[...])

    @pl.when(kk == pl.num_programs(2) - 1)
    def _():
        q_out[...] += bq[...]
        k_out[...] += bk[...]


def _qk_call(x, wq, bq, wk, bk):
    xh, xl = _split(x)
    wqh, wql = _split(wq)
    wkh, wkl = _split(wk)
    grid = (_N // _BM, _H // _BN, _H // _BK)
    return pl.pallas_call(
        _qk_kernel,
        grid=grid,
        in_specs=[
            pl.BlockSpec((_BM, _BK), lambda i, j, k: (i, k)),
            pl.BlockSpec((_BM, _BK), lambda i, j, k: (i, k)),
            pl.BlockSpec((_BN, _BK), lambda i, j, k: (j, k)),
            pl.BlockSpec((_BN, _BK), lambda i, j, k: (j, k)),
            pl.BlockSpec((_BN, _BK), lambda i, j, k: (j, k)),
            pl.BlockSpec((_BN, _BK), lambda i, j, k: (j, k)),
            pl.BlockSpec((1, _BN), lambda i, j, k: (0, j)),
            pl.BlockSpec((1, _BN), lambda i, j, k: (0, j)),
        ],
        out_specs=[
            pl.BlockSpec((_BM, _BN), lambda i, j, k: (i, j)),
            pl.BlockSpec((_BM, _BN), lambda i, j, k: (i, j)),
        ],
        out_shape=[
            jax.ShapeDtypeStruct((_N, _H), jnp.float32),
            jax.ShapeDtypeStruct((_N, _H), jnp.float32),
        ],
        compiler_params=pltpu.CompilerParams(
            dimension_semantics=("parallel", "parallel", "arbitrary"),
        ),
    )(xh, xl, wqh, wql, wkh, wkl, bq.reshape(1, _H), bk.reshape(1, _H))


def _attn_kernel(qh, ql, kh, kl, a_out):
    kk = pl.program_id(2)

    @pl.when(kk == 0)
    def _():
        a_out[...] = jnp.zeros_like(a_out)

    a_out[...] += _mm3(qh[...], ql[...], kh[...], kl[...])

    @pl.when(kk == pl.num_programs(2) - 1)
    def _():
        a_out[...] *= _SCALE


def _attn_call(q, k):
    qh, ql = _split(q)
    kh, kl = _split(k)
    grid = (_N // _BM, _N // _BN, _H // _BK)
    return pl.pallas_call(
        _attn_kernel,
        grid=grid,
        in_specs=[
            pl.BlockSpec((_BM, _BK), lambda i, j, k: (i, k)),
            pl.BlockSpec((_BM, _BK), lambda i, j, k: (i, k)),
            pl.BlockSpec((_BN, _BK), lambda i, j, k: (j, k)),
            pl.BlockSpec((_BN, _BK), lambda i, j, k: (j, k)),
        ],
        out_specs=pl.BlockSpec((_BM, _BN), lambda i, j, k: (i, j)),
        out_shape=jax.ShapeDtypeStruct((_N, _N), jnp.float32),
        compiler_params=pltpu.CompilerParams(
            dimension_semantics=("parallel", "parallel", "arbitrary"),
        ),
    )(qh, ql, kh, kl)


_BR = 256
_RC = 1024
_RROWS = 2048
_NB = _RROWS // _BR
_NCHUNK = _TRI // _RC


def _tu(i):
    return (i * (2 * _N - 1 - i)) // 2


def _tl(p):
    return (p * (p - 1)) // 2


def _unrank(t0, tfun, hi0):
    def body(_, lh):
        lo, hi = lh
        mid = (lo + hi + 1) // 2
        pred = tfun(mid) <= t0
        return (jnp.where(pred, mid, lo), jnp.where(pred, hi, mid - 1))

    lo, _ = jax.lax.fori_loop(0, 11, body, (jnp.int32(0), jnp.int32(hi0)))
    return lo


def _load_unaligned(af, start):
    start_al = pl.multiple_of((start // 128) * 128, 128)
    sh = start % 128
    w = af[:, pl.ds(start_al, _RC + 128)]
    n = _RC + 128
    rolled = pltpu.roll(w, jax.lax.rem(n - sh, n), axis=1)
    return rolled[:, :_RC]


def _flatten_kernel(af, u_o, l_o):
    c = pl.program_id(0)
    t0 = c * _RC
    tvec = t0 + jax.lax.broadcasted_iota(jnp.int32, (1, _RC), 1)

    def u_body(state):
        i, acc = state
        T_i = _tu(i)
        v = _load_unaligned(af, t0 + (i * (_N + 1) + 1 - T_i))
        m = (tvec >= T_i) & (tvec < T_i + (_N - 1 - i))
        return (i + 1, jnp.where(m, v, acc))

    def u_cond(state):
        i, _ = state
        return (i <= _N - 2) & (_tu(i) < t0 + _RC)

    _, uacc = jax.lax.while_loop(
        u_cond, u_body,
        (_unrank(t0, _tu, _N - 2), jnp.zeros((1, _RC), jnp.float32)),
    )
    u_o[...] = uacc.reshape(1, 1, _RC)

    def l_body(state):
        p, acc = state
        T_p = _tl(p)
        v = _load_unaligned(af, t0 + (p * _N - T_p))
        m = (tvec >= T_p) & (tvec < T_p + p)
        return (p + 1, jnp.where(m, v, acc))

    def l_cond(state):
        p, _ = state
        return (p <= _N - 1) & (_tl(p) < t0 + _RC)

    _, lacc = jax.lax.while_loop(
        l_cond, l_body,
        (_unrank(t0, _tl, _N - 1), jnp.zeros((1, _RC), jnp.float32)),
    )
    l_o[...] = lacc.reshape(1, 1, _RC)


def _flatten_call(af):
    return pl.pallas_call(
        _flatten_kernel,
        grid=(_NCHUNK,),
        in_specs=[pl.BlockSpec(memory_space=pltpu.VMEM)],
        out_specs=[
            pl.BlockSpec((1, 1, _RC), lambda c: (c, 0, 0)),
            pl.BlockSpec((1, 1, _RC), lambda c: (c, 0, 0)),
        ],
        out_shape=[
            jax.ShapeDtypeStruct((_NCHUNK, 1, _RC), jnp.float32),
            jax.ShapeDtypeStruct((_NCHUNK, 1, _RC), jnp.float32),
        ],
        compiler_params=pltpu.CompilerParams(
            dimension_semantics=("arbitrary",),
            vmem_limit_bytes=100 * 1024 * 1024,
        ),
    )(af)


def _reduce_kernel(u, l, logp_o, r_o, c_o, m_s, s_s, vals_s, t_s):
    b = pl.program_id(0)
    lane = jax.lax.broadcasted_iota(jnp.int32, (1, 128), 1)

    @pl.when(b == 0)
    def _():
        m_s[0, 0] = _NINF
        s_s[0, 0] = 0.0
        vals_s[...] = jnp.full((1, 128), _NINF, jnp.float32)
        t_s[...] = jnp.full((1, 128), _BIG, jnp.int32)

    v = u[...] + l[...]
    rows = jax.lax.broadcasted_iota(jnp.int32, (_BR, _RC), 0)
    cols = jax.lax.broadcasted_iota(jnp.int32, (_BR, _RC), 1)
    tg = (b * _BR + rows) * _RC + cols

    bm = jnp.max(v)
    mo = m_s[0, 0]
    mn = jnp.maximum(mo, bm)
    s_s[0, 0] = s_s[0, 0] * jnp.exp(mo - mn) + jnp.sum(jnp.exp(v - mn))
    m_s[0, 0] = mn

    cand_v = jnp.full((1, 128), _NINF, jnp.float32)
    cand_t = jnp.full((1, 128), _BIG, jnp.int32)
    vv = v
    for it in range(_TOPK):
        mv = jnp.max(vv)
        ts = jnp.min(jnp.where(vv == mv, tg, _BIG))
        cand_v = jnp.where(lane == _TOPK + it, mv, cand_v)
        cand_t = jnp.where(lane == _TOPK + it, ts, cand_t)
        vv = jnp.where(tg == ts, _NINF, vv)

    allv = jnp.where(lane < _TOPK, vals_s[...], cand_v)
    allt = jnp.where(lane < _TOPK, t_s[...], cand_t)
    nv = jnp.full((1, 128), _NINF, jnp.float32)
    nt = jnp.full((1, 128), _BIG, jnp.int32)
    for slot in range(_TOPK):
        mv = jnp.max(allv)
        ts = jnp.min(jnp.where(allv == mv, allt, _BIG))
        nv = jnp.where(lane == slot, mv, nv)
        nt = jnp.where(lane == slot, ts, nt)
        allv = jnp.where(allt == ts, _NINF, allv)
    vals_s[...] = nv
    t_s[...] = nt

    @pl.when(b == _NB - 1)
    def _():
        lse = m_s[0, 0] + jnp.log(s_s[0, 0])
        t = t_s[...]
        lo = jnp.zeros((1, 128), jnp.int32)
        hi = jnp.full((1, 128), _N - 2, jnp.int32)
        for _it in range(11):
            mid = (lo + hi + 1) // 2
            tu = (mid * (2 * _N - 1 - mid)) // 2
            pred = tu <= t
            lo = jnp.where(pred, mid, lo)
            hi = jnp.where(pred, hi, mid - 1)
        tu_lo = (lo * (2 * _N - 1 - lo)) // 2
        logp_o[...] = vals_s[...] - lse
        r_o[...] = lo
        c_o[...] = t - tu_lo + lo + 1


def _reduce_call(u2, l2):
    return pl.pallas_call(
        _reduce_kernel,
        grid=(_NB,),
        in_specs=[
            pl.BlockSpec((_BR, _RC), lambda b: (b, 0)),
            pl.BlockSpec((_BR, _RC), lambda b: (b, 0)),
        ],
        out_specs=[
            pl.BlockSpec((1, 128), lambda b: (0, 0)),
            pl.BlockSpec((1, 128), lambda b: (0, 0)),
            pl.BlockSpec((1, 128), lambda b: (0, 0)),
        ],
        out_shape=[
            jax.ShapeDtypeStruct((1, 128), jnp.float32),
            jax.ShapeDtypeStruct((1, 128), jnp.int32),
            jax.ShapeDtypeStruct((1, 128), jnp.int32),
        ],
        scratch_shapes=[
            pltpu.SMEM((1, 1), jnp.float32),
            pltpu.SMEM((1, 1), jnp.float32),
            pltpu.VMEM((1, 128), jnp.float32),
            pltpu.VMEM((1, 128), jnp.int32),
        ],
        compiler_params=pltpu.CompilerParams(
            dimension_semantics=("arbitrary",),
        ),
    )(u2, l2)


def kernel(clause_emb, Wq_w, Wq_b, Wk_w, Wk_b, keep_mask):
    x0 = clause_emb[0]
    q, k = _qk_call(x0, Wq_w, Wq_b, Wk_w, Wk_b)
    a = _attn_call(q, k)

    af = jnp.concatenate(
        [a.reshape(1, _N * _N), jnp.zeros((1, _N), jnp.float32)], axis=1
    )
    uf, lf = _flatten_call(af)

    ninf_row = jnp.full((1, _RC), _NINF, jnp.float32)
    u2 = jnp.concatenate([uf.reshape(_NCHUNK, _RC), ninf_row], axis=0)
    l2 = jnp.concatenate([lf.reshape(_NCHUNK, _RC), ninf_row], axis=0)
    logp, rr, cc = _reduce_call(u2, l2)
    return (logp[0, :_TOPK], rr[0, :_TOPK], cc[0, :_TOPK])


# speedup vs baseline: 56.8028x; 4.0242x over previous
import functools
import math

import jax
import jax.numpy as jnp
from jax import lax
from jax.experimental import pallas as pl
from jax.experimental.pallas import tpu as pltpu
from jax.experimental.pallas import tpu_sc as plsc

_N = 2048
_H = 2048
_TOPK = 8
_TRI = _N * (_N - 1) // 2
_SCALE = 1.0 / math.sqrt(_H)
_BIG = 2**30
_NINF = float("-inf")

_BM = 512
_BN = 512
_BK = 512

_DIMS = (((1,), (1,)), ((), ()))


def _mm3(ah, al, bh, bl):
    return (
        jax.lax.dot_general(ah, bh, _DIMS, preferred_element_type=jnp.float32)
        + jax.lax.dot_general(ah, bl, _DIMS, preferred_element_type=jnp.float32)
        + jax.lax.dot_general(al, bh, _DIMS, preferred_element_type=jnp.float32)
    )


def _split(x):
    hi = x.astype(jnp.bfloat16)
    lo = (x - hi.astype(jnp.float32)).astype(jnp.bfloat16)
    return hi, lo


def _qk_kernel(xh, xl, wqh, wql, wkh, wkl, bq, bk, q_out, k_out):
    kk = pl.program_id(2)

    @pl.when(kk == 0)
    def _():
        q_out[...] = jnp.zeros_like(q_out)
        k_out[...] = jnp.zeros_like(k_out)

    xh_, xl_ = xh[...], xl[...]
    q_out[...] += _mm3(xh_, xl_, wqh[...], wql[...])
    k_out[...] += _mm3(xh_, xl_, wkh[...], wkl[...])

    @pl.when(kk == pl.num_programs(2) - 1)
    def _():
        q_out[...] += bq[...]
        k_out[...] += bk[...]


def _qk_call(x, wq, bq, wk, bk):
    xh, xl = _split(x)
    wqh, wql = _split(wq)
    wkh, wkl = _split(wk)
    grid = (_N // _BM, _H // _BN, _H // _BK)
    return pl.pallas_call(
        _qk_kernel,
        grid=grid,
        in_specs=[
            pl.BlockSpec((_BM, _BK), lambda i, j, k: (i, k)),
            pl.BlockSpec((_BM, _BK), lambda i, j, k: (i, k)),
            pl.BlockSpec((_BN, _BK), lambda i, j, k: (j, k)),
            pl.BlockSpec((_BN, _BK), lambda i, j, k: (j, k)),
            pl.BlockSpec((_BN, _BK), lambda i, j, k: (j, k)),
            pl.BlockSpec((_BN, _BK), lambda i, j, k: (j, k)),
            pl.BlockSpec((1, _BN), lambda i, j, k: (0, j)),
            pl.BlockSpec((1, _BN), lambda i, j, k: (0, j)),
        ],
        out_specs=[
            pl.BlockSpec((_BM, _BN), lambda i, j, k: (i, j)),
            pl.BlockSpec((_BM, _BN), lambda i, j, k: (i, j)),
        ],
        out_shape=[
            jax.ShapeDtypeStruct((_N, _H), jnp.float32),
            jax.ShapeDtypeStruct((_N, _H), jnp.float32),
        ],
        compiler_params=pltpu.CompilerParams(
            dimension_semantics=("parallel", "parallel", "arbitrary"),
        ),
    )(xh, xl, wqh, wql, wkh, wkl, bq.reshape(1, _H), bk.reshape(1, _H))


def _attn_kernel(qh, ql, kh, kl, a_out):
    kk = pl.program_id(2)

    @pl.when(kk == 0)
    def _():
        a_out[...] = jnp.zeros_like(a_out)

    a_out[...] += _mm3(qh[...], ql[...], kh[...], kl[...])

    @pl.when(kk == pl.num_programs(2) - 1)
    def _():
        a_out[...] *= _SCALE


def _attn_call(q, k):
    qh, ql = _split(q)
    kh, kl = _split(k)
    grid = (_N // _BM, _N // _BN, _H // _BK)
    return pl.pallas_call(
        _attn_kernel,
        grid=grid,
        in_specs=[
            pl.BlockSpec((_BM, _BK), lambda i, j, k: (i, k)),
            pl.BlockSpec((_BM, _BK), lambda i, j, k: (i, k)),
            pl.BlockSpec((_BN, _BK), lambda i, j, k: (j, k)),
            pl.BlockSpec((_BN, _BK), lambda i, j, k: (j, k)),
        ],
        out_specs=pl.BlockSpec((_BM, _BN), lambda i, j, k: (i, j)),
        out_shape=jax.ShapeDtypeStruct((_N, _N), jnp.float32),
        compiler_params=pltpu.CompilerParams(
            dimension_semantics=("parallel", "parallel", "arbitrary"),
        ),
    )(qh, ql, kh, kl)


_BR = 256
_RC = 1024
_RROWS = 2048
_NB = _RROWS // _BR
_NCHUNK = _TRI // _RC


def _tu(i):
    return (i * (2 * _N - 1 - i)) // 2


def _tl(p):
    return (p * (p - 1)) // 2


def _unrank(t0, tfun, hi0):
    def body(_, lh):
        lo, hi = lh
        mid = (lo + hi + 1) // 2
        pred = tfun(mid) <= t0
        return (jnp.where(pred, mid, lo), jnp.where(pred, hi, mid - 1))

    lo, _ = jax.lax.fori_loop(0, 11, body, (jnp.int32(0), jnp.int32(hi0)))
    return lo


def _load_unaligned(af, start):
    start_al = pl.multiple_of((start // 128) * 128, 128)
    sh = start % 128
    w = af[:, pl.ds(start_al, _RC + 128)]
    n = _RC + 128
    rolled = pltpu.roll(w, jax.lax.rem(n - sh, n), axis=1)
    return rolled[:, :_RC]


def _flatten_kernel(af, u_o, l_o):
    c = pl.program_id(0)
    t0 = c * _RC
    tvec = t0 + jax.lax.broadcasted_iota(jnp.int32, (1, _RC), 1)

    def u_body(state):
        i, acc = state
        T_i = _tu(i)
        v = _load_unaligned(af, t0 + (i * (_N + 1) + 1 - T_i))
        m = (tvec >= T_i) & (tvec < T_i + (_N - 1 - i))
        return (i + 1, jnp.where(m, v, acc))

    def u_cond(state):
        i, _ = state
        return (i <= _N - 2) & (_tu(i) < t0 + _RC)

    _, uacc = jax.lax.while_loop(
        u_cond, u_body,
        (_unrank(t0, _tu, _N - 2), jnp.zeros((1, _RC), jnp.float32)),
    )
    u_o[...] = uacc.reshape(1, 1, _RC)

    def l_body(state):
        p, acc = state
        T_p = _tl(p)
        v = _load_unaligned(af, t0 + (p * _N - T_p))
        m = (tvec >= T_p) & (tvec < T_p + p)
        return (p + 1, jnp.where(m, v, acc))

    def l_cond(state):
        p, _ = state
        return (p <= _N - 1) & (_tl(p) < t0 + _RC)

    _, lacc = jax.lax.while_loop(
        l_cond, l_body,
        (_unrank(t0, _tl, _N - 1), jnp.zeros((1, _RC), jnp.float32)),
    )
    l_o[...] = lacc.reshape(1, 1, _RC)


_NW = 32
_TW = _TRI // _NW
_SG = 16


def _sc_unrank(t0, tfun, hi0):
    def body(_, lh):
        lo, hi = lh
        mid = (lo + hi + 1) // 2
        pred = tfun(mid) <= t0
        return (jnp.where(pred, mid, lo), jnp.where(pred, hi, mid - 1))

    lo, _ = lax.fori_loop(0, 11, body, (jnp.int32(0), jnp.int32(hi0)))
    return lo


def _sc_phase(a_hbm, stage, build, out_hbm, t0, t1, tfun, lenfun, colfun, rmax):
    lane = jnp.arange(16, dtype=jnp.int32)

    r_start = _sc_unrank(t0, tfun, rmax)
    r_end = jnp.minimum(_sc_unrank(t1 - 1, tfun, rmax) + 1, rmax + 1)
    b0 = (r_start // 8) * 8
    n_groups = (r_end - b0 + _SG - 1) // _SG

    def group_body(g, _):
        b_raw = b0 + g * _SG
        b = pl.multiple_of(jnp.minimum(b_raw, _N - _SG), 8)
        pltpu.sync_copy(a_hbm.at[pl.ds(b * _N, _SG * _N)],
                        stage.at[pl.ds(0, _SG * _N)])
        row_lo = jnp.maximum(r_start, b_raw)
        row_hi = jnp.minimum(r_end, b_raw + _SG)

        def row_body(rr, _):
            T_r = tfun(rr)
            s = jnp.maximum(T_r, t0)
            e = jnp.minimum(T_r + lenfun(rr), t1)
            c0 = colfun(rr, s)
            d0 = s - t0
            nchunks = (jnp.maximum(e - s, 0) + 15) // 16
            src0 = (rr - b) * _N + c0

            def copy_body(kk, _):
                off = kk * 16
                build[pl.ds(d0 + off, 16)] = stage[pl.ds(src0 + off, 16)]
                return 0

            lax.fori_loop(0, nchunks, copy_body, 0, unroll=False)
            return 0

        lax.fori_loop(row_lo, row_hi, row_body, 0, unroll=False)
        return 0

    lax.fori_loop(0, n_groups, group_body, 0, unroll=False)
    pltpu.sync_copy(build.at[pl.ds(0, _TW)], out_hbm.at[pl.ds(t0, _TW)])


def _sc_flatten_call(a):
    mesh = plsc.VectorSubcoreMesh(core_axis_name="c", subcore_axis_name="s")

    @functools.partial(
        pl.kernel,
        out_type=[
            jax.ShapeDtypeStruct((_TRI,), jnp.float32),
            jax.ShapeDtypeStruct((_TRI,), jnp.float32),
        ],
        mesh=mesh,
        scratch_types=[
            pltpu.VMEM((_SG * _N + 16,), jnp.float32),
            pltpu.VMEM((_TW + 16,), jnp.float32),
        ],
    )
    def k(a_hbm, u_hbm, l_hbm, stage, build):
        wid = lax.axis_index("s") * 2 + lax.axis_index("c")
        t0 = wid * _TW
        t1 = t0 + _TW
        _sc_phase(
            a_hbm, stage, build, u_hbm, t0, t1,
            _tu, lambda i: _N - 1 - i,
            lambda i, s: i + 1 + (s - _tu(i)), _N - 2,
        )
        _sc_phase(
            a_hbm, stage, build, l_hbm, t0, t1,
            _tl, lambda p: p,
            lambda p, s: s - _tl(p), _N - 1,
        )

    return k(a.reshape(_N * _N))


def _flatten_call(af):
    return pl.pallas_call(
        _flatten_kernel,
        grid=(_NCHUNK,),
        in_specs=[pl.BlockSpec(memory_space=pltpu.VMEM)],
        out_specs=[
            pl.BlockSpec((1, 1, _RC), lambda c: (c, 0, 0)),
            pl.BlockSpec((1, 1, _RC), lambda c: (c, 0, 0)),
        ],
        out_shape=[
            jax.ShapeDtypeStruct((_NCHUNK, 1, _RC), jnp.float32),
            jax.ShapeDtypeStruct((_NCHUNK, 1, _RC), jnp.float32),
        ],
        compiler_params=pltpu.CompilerParams(
            dimension_semantics=("arbitrary",),
            vmem_limit_bytes=100 * 1024 * 1024,
        ),
    )(af)


def _reduce_kernel(u, l, logp_o, r_o, c_o, m_s, s_s, vals_s, t_s):
    b = pl.program_id(0)
    lane = jax.lax.broadcasted_iota(jnp.int32, (1, 128), 1)

    @pl.when(b == 0)
    def _():
        m_s[0, 0] = _NINF
        s_s[0, 0] = 0.0
        vals_s[...] = jnp.full((1, 128), _NINF, jnp.float32)
        t_s[...] = jnp.full((1, 128), _BIG, jnp.int32)

    v = u[...] + l[...]
    rows = jax.lax.broadcasted_iota(jnp.int32, (_BR, _RC), 0)
    cols = jax.lax.broadcasted_iota(jnp.int32, (_BR, _RC), 1)
    tg = (b * _BR + rows) * _RC + cols

    bm = jnp.max(v)
    mo = m_s[0, 0]
    mn = jnp.maximum(mo, bm)
    s_s[0, 0] = s_s[0, 0] * jnp.exp(mo - mn) + jnp.sum(jnp.exp(v - mn))
    m_s[0, 0] = mn

    cand_v = jnp.full((1, 128), _NINF, jnp.float32)
    cand_t = jnp.full((1, 128), _BIG, jnp.int32)
    vv = v
    for it in range(_TOPK):
        mv = jnp.max(vv)
        ts = jnp.min(jnp.where(vv == mv, tg, _BIG))
        cand_v = jnp.where(lane == _TOPK + it, mv, cand_v)
        cand_t = jnp.where(lane == _TOPK + it, ts, cand_t)
        vv = jnp.where(tg == ts, _NINF, vv)

    allv = jnp.where(lane < _TOPK, vals_s[...], cand_v)
    allt = jnp.where(lane < _TOPK, t_s[...], cand_t)
    nv = jnp.full((1, 128), _NINF, jnp.float32)
    nt = jnp.full((1, 128), _BIG, jnp.int32)
    for slot in range(_TOPK):
        mv = jnp.max(allv)
        ts = jnp.min(jnp.where(allv == mv, allt, _BIG))
        nv = jnp.where(lane == slot, mv, nv)
        nt = jnp.where(lane == slot, ts, nt)
        allv = jnp.where(allt == ts, _NINF, allv)
    vals_s[...] = nv
    t_s[...] = nt

    @pl.when(b == _NB - 1)
    def _():
        lse = m_s[0, 0] + jnp.log(s_s[0, 0])
        t = t_s[...]
        lo = jnp.zeros((1, 128), jnp.int32)
        hi = jnp.full((1, 128), _N - 2, jnp.int32)
        for _it in range(11):
            mid = (lo + hi + 1) // 2
            tu = (mid * (2 * _N - 1 - mid)) // 2
            pred = tu <= t
            lo = jnp.where(pred, mid, lo)
            hi = jnp.where(pred, hi, mid - 1)
        tu_lo = (lo * (2 * _N - 1 - lo)) // 2
        logp_o[...] = vals_s[...] - lse
        r_o[...] = lo
        c_o[...] = t - tu_lo + lo + 1


def _reduce_call(u2, l2):
    return pl.pallas_call(
        _reduce_kernel,
        grid=(_NB,),
        in_specs=[
            pl.BlockSpec((_BR, _RC), lambda b: (b, 0)),
            pl.BlockSpec((_BR, _RC), lambda b: (b, 0)),
        ],
        out_specs=[
            pl.BlockSpec((1, 128), lambda b: (0, 0)),
            pl.BlockSpec((1, 128), lambda b: (0, 0)),
            pl.BlockSpec((1, 128), lambda b: (0, 0)),
        ],
        out_shape=[
            jax.ShapeDtypeStruct((1, 128), jnp.float32),
            jax.ShapeDtypeStruct((1, 128), jnp.int32),
            jax.ShapeDtypeStruct((1, 128), jnp.int32),
        ],
        scratch_shapes=[
            pltpu.SMEM((1, 1), jnp.float32),
            pltpu.SMEM((1, 1), jnp.float32),
            pltpu.VMEM((1, 128), jnp.float32),
            pltpu.VMEM((1, 128), jnp.int32),
        ],
        compiler_params=pltpu.CompilerParams(
            dimension_semantics=("arbitrary",),
        ),
    )(u2, l2)


def kernel(clause_emb, Wq_w, Wq_b, Wk_w, Wk_b, keep_mask):
    x0 = clause_emb[0]
    q, k = _qk_call(x0, Wq_w, Wq_b, Wk_w, Wk_b)
    a = _attn_call(q, k)

    uf, lf = _sc_flatten_call(a)

    ninf_row = jnp.full((1, _RC), _NINF, jnp.float32)
    u2 = jnp.concatenate([uf.reshape(_NCHUNK, _RC), ninf_row], axis=0)
    l2 = jnp.concatenate([lf.reshape(_NCHUNK, _RC), ninf_row], axis=0)
    logp, rr, cc = _reduce_call(u2, l2)
    return (logp[0, :_TOPK], rr[0, :_TOPK], cc[0, :_TOPK])


# speedup vs baseline: 68.4307x; 1.2047x over previous
import functools
import math

import jax
import jax.numpy as jnp
from jax import lax
from jax.experimental import pallas as pl
from jax.experimental.pallas import tpu as pltpu
from jax.experimental.pallas import tpu_sc as plsc

_N = 2048
_H = 2048
_TOPK = 8
_TRI = _N * (_N - 1) // 2
_SCALE = 1.0 / math.sqrt(_H)
_BIG = 2**30
_NINF = float("-inf")

_BM = 512
_BN = 512
_BK = 512

_DIMS = (((1,), (1,)), ((), ()))


def _mm3(ah, al, bh, bl):
    return (
        jax.lax.dot_general(ah, bh, _DIMS, preferred_element_type=jnp.float32)
        + jax.lax.dot_general(ah, bl, _DIMS, preferred_element_type=jnp.float32)
        + jax.lax.dot_general(al, bh, _DIMS, preferred_element_type=jnp.float32)
    )


def _split(x):
    hi = x.astype(jnp.bfloat16)
    lo = (x - hi.astype(jnp.float32)).astype(jnp.bfloat16)
    return hi, lo


def _qk_kernel(x3, wq, wk, bq, bk, q_out, k_out):
    kk = pl.program_id(2)

    @pl.when(kk == 0)
    def _():
        q_out[...] = jnp.zeros_like(q_out)
        k_out[...] = jnp.zeros_like(k_out)

    xh, xl = _split(x3[0])
    wqh, wql = _split(wq[...])
    wkh, wkl = _split(wk[...])
    q_out[...] += _mm3(xh, xl, wqh, wql)
    k_out[...] += _mm3(xh, xl, wkh, wkl)

    @pl.when(kk == pl.num_programs(2) - 1)
    def _():
        q_out[...] += bq[...]
        k_out[...] += bk[...]


def _qk_call(clause_emb, wq, bq, wk, bk):
    grid = (_N // _BM, _H // _BN, _H // _BK)
    return pl.pallas_call(
        _qk_kernel,
        grid=grid,
        in_specs=[
            pl.BlockSpec((1, _BM, _BK), lambda i, j, k: (0, i, k)),
            pl.BlockSpec((_BN, _BK), lambda i, j, k: (j, k)),
            pl.BlockSpec((_BN, _BK), lambda i, j, k: (j, k)),
            pl.BlockSpec((1, _BN), lambda i, j, k: (0, j)),
            pl.BlockSpec((1, _BN), lambda i, j, k: (0, j)),
        ],
        out_specs=[
            pl.BlockSpec((_BM, _BN), lambda i, j, k: (i, j)),
            pl.BlockSpec((_BM, _BN), lambda i, j, k: (i, j)),
        ],
        out_shape=[
            jax.ShapeDtypeStruct((_N, _H), jnp.float32),
            jax.ShapeDtypeStruct((_N, _H), jnp.float32),
        ],
        compiler_params=pltpu.CompilerParams(
            dimension_semantics=("parallel", "parallel", "arbitrary"),
        ),
    )(clause_emb, wq, wk, bq.reshape(1, _H), bk.reshape(1, _H))


def _attn_kernel(q, k, a_out):
    kk = pl.program_id(2)

    @pl.when(kk == 0)
    def _():
        a_out[...] = jnp.zeros_like(a_out)

    qh, ql = _split(q[...])
    kh, kl = _split(k[...])
    a_out[...] += _mm3(qh, ql, kh, kl)

    @pl.when(kk == pl.num_programs(2) - 1)
    def _():
        a_out[...] *= _SCALE


def _attn_call(q, k):
    grid = (_N // _BM, _N // _BN, _H // _BK)
    return pl.pallas_call(
        _attn_kernel,
        grid=grid,
        in_specs=[
            pl.BlockSpec((_BM, _BK), lambda i, j, k: (i, k)),
            pl.BlockSpec((_BN, _BK), lambda i, j, k: (j, k)),
        ],
        out_specs=pl.BlockSpec((_BM, _BN), lambda i, j, k: (i, j)),
        out_shape=jax.ShapeDtypeStruct((_N, _N), jnp.float32),
        compiler_params=pltpu.CompilerParams(
            dimension_semantics=("parallel", "parallel", "arbitrary"),
        ),
    )(q, k)


_BR = 256
_RC = 1024
_RROWS = 2048
_NB = _RROWS // _BR
_NCHUNK = _TRI // _RC


def _tu(i):
    return (i * (2 * _N - 1 - i)) // 2


def _tl(p):
    return (p * (p - 1)) // 2


def _unrank(t0, tfun, hi0):
    def body(_, lh):
        lo, hi = lh
        mid = (lo + hi + 1) // 2
        pred = tfun(mid) <= t0
        return (jnp.where(pred, mid, lo), jnp.where(pred, hi, mid - 1))

    lo, _ = jax.lax.fori_loop(0, 11, body, (jnp.int32(0), jnp.int32(hi0)))
    return lo


def _load_unaligned(af, start):
    start_al = pl.multiple_of((start // 128) * 128, 128)
    sh = start % 128
    w = af[:, pl.ds(start_al, _RC + 128)]
    n = _RC + 128
    rolled = pltpu.roll(w, jax.lax.rem(n - sh, n), axis=1)
    return rolled[:, :_RC]


def _flatten_kernel(af, u_o, l_o):
    c = pl.program_id(0)
    t0 = c * _RC
    tvec = t0 + jax.lax.broadcasted_iota(jnp.int32, (1, _RC), 1)

    def u_body(state):
        i, acc = state
        T_i = _tu(i)
        v = _load_unaligned(af, t0 + (i * (_N + 1) + 1 - T_i))
        m = (tvec >= T_i) & (tvec < T_i + (_N - 1 - i))
        return (i + 1, jnp.where(m, v, acc))

    def u_cond(state):
        i, _ = state
        return (i <= _N - 2) & (_tu(i) < t0 + _RC)

    _, uacc = jax.lax.while_loop(
        u_cond, u_body,
        (_unrank(t0, _tu, _N - 2), jnp.zeros((1, _RC), jnp.float32)),
    )
    u_o[...] = uacc.reshape(1, 1, _RC)

    def l_body(state):
        p, acc = state
        T_p = _tl(p)
        v = _load_unaligned(af, t0 + (p * _N - T_p))
        m = (tvec >= T_p) & (tvec < T_p + p)
        return (p + 1, jnp.where(m, v, acc))

    def l_cond(state):
        p, _ = state
        return (p <= _N - 1) & (_tl(p) < t0 + _RC)

    _, lacc = jax.lax.while_loop(
        l_cond, l_body,
        (_unrank(t0, _tl, _N - 1), jnp.zeros((1, _RC), jnp.float32)),
    )
    l_o[...] = lacc.reshape(1, 1, _RC)


_NW = 32
_TW = _TRI // _NW
_SG = 16


def _sc_unrank(t0, tfun, hi0):
    def body(_, lh):
        lo, hi = lh
        mid = (lo + hi + 1) // 2
        pred = tfun(mid) <= t0
        return (jnp.where(pred, mid, lo), jnp.where(pred, hi, mid - 1))

    lo, _ = lax.fori_loop(0, 11, body, (jnp.int32(0), jnp.int32(hi0)))
    return lo


def _sc_phase(a_hbm, stage, build, out_hbm, t0, t1, tfun, lenfun, colfun, rmax):
    lane = jnp.arange(16, dtype=jnp.int32)

    r_start = _sc_unrank(t0, tfun, rmax)
    r_end = jnp.minimum(_sc_unrank(t1 - 1, tfun, rmax) + 1, rmax + 1)
    b0 = (r_start // 8) * 8
    n_groups = (r_end - b0 + _SG - 1) // _SG

    def group_body(g, _):
        b_raw = b0 + g * _SG
        b = pl.multiple_of(jnp.minimum(b_raw, _N - _SG), 8)
        pltpu.sync_copy(a_hbm.at[pl.ds(b * _N, _SG * _N)],
                        stage.at[pl.ds(0, _SG * _N)])
        row_lo = jnp.maximum(r_start, b_raw)
        row_hi = jnp.minimum(r_end, b_raw + _SG)

        def row_body(rr, _):
            T_r = tfun(rr)
            s = jnp.maximum(T_r, t0)
            e = jnp.minimum(T_r + lenfun(rr), t1)
            c0 = colfun(rr, s)
            d0 = s - t0
            nchunks = (jnp.maximum(e - s, 0) + 15) // 16
            src0 = (rr - b) * _N + c0

            def copy_body(kk, _):
                off = kk * 16
                build[pl.ds(d0 + off, 16)] = stage[pl.ds(src0 + off, 16)]
                return 0

            lax.fori_loop(0, nchunks, copy_body, 0, unroll=False)
            return 0

        lax.fori_loop(row_lo, row_hi, row_body, 0, unroll=False)
        return 0

    lax.fori_loop(0, n_groups, group_body, 0, unroll=False)
    pltpu.sync_copy(build.at[pl.ds(0, _TW)], out_hbm.at[pl.ds(t0, _TW)])


def _sc_flatten_call(a):
    mesh = plsc.VectorSubcoreMesh(core_axis_name="c", subcore_axis_name="s")

    @functools.partial(
        pl.kernel,
        out_type=[
            jax.ShapeDtypeStruct((_TRI,), jnp.float32),
            jax.ShapeDtypeStruct((_TRI,), jnp.float32),
        ],
        mesh=mesh,
        scratch_types=[
            pltpu.VMEM((_SG * _N + 16,), jnp.float32),
            pltpu.VMEM((_TW + 16,), jnp.float32),
        ],
    )
    def k(a_hbm, u_hbm, l_hbm, stage, build):
        wid = lax.axis_index("s") * 2 + lax.axis_index("c")
        t0 = wid * _TW
        t1 = t0 + _TW
        _sc_phase(
            a_hbm, stage, build, u_hbm, t0, t1,
            _tu, lambda i: _N - 1 - i,
            lambda i, s: i + 1 + (s - _tu(i)), _N - 2,
        )
        _sc_phase(
            a_hbm, stage, build, l_hbm, t0, t1,
            _tl, lambda p: p,
            lambda p, s: s - _tl(p), _N - 1,
        )

    return k(a.reshape(_N * _N))


def _flatten_call(af):
    return pl.pallas_call(
        _flatten_kernel,
        grid=(_NCHUNK,),
        in_specs=[pl.BlockSpec(memory_space=pltpu.VMEM)],
        out_specs=[
            pl.BlockSpec((1, 1, _RC), lambda c: (c, 0, 0)),
            pl.BlockSpec((1, 1, _RC), lambda c: (c, 0, 0)),
        ],
        out_shape=[
            jax.ShapeDtypeStruct((_NCHUNK, 1, _RC), jnp.float32),
            jax.ShapeDtypeStruct((_NCHUNK, 1, _RC), jnp.float32),
        ],
        compiler_params=pltpu.CompilerParams(
            dimension_semantics=("arbitrary",),
            vmem_limit_bytes=100 * 1024 * 1024,
        ),
    )(af)


def _reduce_kernel(u, l, logp_o, r_o, c_o, m_s, s_s, th_s, vals_s, t_s):
    b = pl.program_id(0)
    lane = jax.lax.broadcasted_iota(jnp.int32, (1, 128), 1)

    @pl.when(b == 0)
    def _():
        m_s[0, 0] = _NINF
        s_s[0, 0] = 0.0
        th_s[0, 0] = _NINF
        vals_s[...] = jnp.full((1, 128), _NINF, jnp.float32)
        t_s[...] = jnp.full((1, 128), _BIG, jnp.int32)

    rows = jax.lax.broadcasted_iota(jnp.int32, (_BR, _RC), 0)
    cols = jax.lax.broadcasted_iota(jnp.int32, (_BR, _RC), 1)
    tg = (b * _BR + rows) * _RC + cols
    v = jnp.where(tg < _TRI, u[...] + l[...], _NINF)

    bm = jnp.max(v)
    mo = m_s[0, 0]
    mn = jnp.maximum(mo, bm)
    s_s[0, 0] = s_s[0, 0] * jnp.exp(mo - mn) + jnp.sum(jnp.exp(v - mn))
    m_s[0, 0] = mn

    @pl.when(bm > th_s[0, 0])
    def _():
        cand_v = jnp.full((1, 128), _NINF, jnp.float32)
        cand_t = jnp.full((1, 128), _BIG, jnp.int32)
        vv = v
        for it in range(_TOPK):
            mv = jnp.max(vv)
            ts = jnp.min(jnp.where(vv == mv, tg, _BIG))
            cand_v = jnp.where(lane == _TOPK + it, mv, cand_v)
            cand_t = jnp.where(lane == _TOPK + it, ts, cand_t)
            vv = jnp.where(tg == ts, _NINF, vv)

        allv = jnp.where(lane < _TOPK, vals_s[...], cand_v)
        allt = jnp.where(lane < _TOPK, t_s[...], cand_t)
        nv = jnp.full((1, 128), _NINF, jnp.float32)
        nt = jnp.full((1, 128), _BIG, jnp.int32)
        for slot in range(_TOPK):
            mv = jnp.max(allv)
            ts = jnp.min(jnp.where(allv == mv, allt, _BIG))
            nv = jnp.where(lane == slot, mv, nv)
            nt = jnp.where(lane == slot, ts, nt)
            allv = jnp.where(allt == ts, _NINF, allv)
        vals_s[...] = nv
        t_s[...] = nt
        th_s[0, 0] = jnp.min(jnp.where(lane < _TOPK, nv, jnp.inf))

    @pl.when(b == _NB - 1)
    def _():
        lse = m_s[0, 0] + jnp.log(s_s[0, 0])
        t = t_s[...]
        lo = jnp.zeros((1, 128), jnp.int32)
        hi = jnp.full((1, 128), _N - 2, jnp.int32)
        for _it in range(11):
            mid = (lo + hi + 1) // 2
            tu = (mid * (2 * _N - 1 - mid)) // 2
            pred = tu <= t
            lo = jnp.where(pred, mid, lo)
            hi = jnp.where(pred, hi, mid - 1)
        tu_lo = (lo * (2 * _N - 1 - lo)) // 2
        logp_o[...] = vals_s[...] - lse
        r_o[...] = lo
        c_o[...] = t - tu_lo + lo + 1


def _reduce_call(u2, l2):
    return pl.pallas_call(
        _reduce_kernel,
        grid=(_NB,),
        in_specs=[
            pl.BlockSpec((_BR, _RC), lambda b: (b, 0)),
            pl.BlockSpec((_BR, _RC), lambda b: (b, 0)),
        ],
        out_specs=[
            pl.BlockSpec((1, 128), lambda b: (0, 0)),
            pl.BlockSpec((1, 128), lambda b: (0, 0)),
            pl.BlockSpec((1, 128), lambda b: (0, 0)),
        ],
        out_shape=[
            jax.ShapeDtypeStruct((1, 128), jnp.float32),
            jax.ShapeDtypeStruct((1, 128), jnp.int32),
            jax.ShapeDtypeStruct((1, 128), jnp.int32),
        ],
        scratch_shapes=[
            pltpu.SMEM((1, 1), jnp.float32),
            pltpu.SMEM((1, 1), jnp.float32),
            pltpu.SMEM((1, 1), jnp.float32),
            pltpu.VMEM((1, 128), jnp.float32),
            pltpu.VMEM((1, 128), jnp.int32),
        ],
        compiler_params=pltpu.CompilerParams(
            dimension_semantics=("arbitrary",),
        ),
    )(u2, l2)


def kernel(clause_emb, Wq_w, Wq_b, Wk_w, Wk_b, keep_mask):
    q, k = _qk_call(clause_emb, Wq_w, Wq_b, Wk_w, Wk_b)
    a = _attn_call(q, k)
    uf, lf = _sc_flatten_call(a)
    logp, rr, cc = _reduce_call(
        uf.reshape(_NCHUNK, _RC), lf.reshape(_NCHUNK, _RC)
    )
    return (logp[0, :_TOPK], rr[0, :_TOPK], cc[0, :_TOPK])


# speedup vs baseline: 78.5612x; 1.1480x over previous
import functools
import math

import jax
import jax.numpy as jnp
from jax import lax
from jax.experimental import pallas as pl
from jax.experimental.pallas import tpu as pltpu
from jax.experimental.pallas import tpu_sc as plsc

_N = 2048
_H = 2048
_TOPK = 8
_TRI = _N * (_N - 1) // 2
_SCALE = 1.0 / math.sqrt(_H)
_BIG = 2**30
_NINF = float("-inf")

_BM = 512
_BN = 512
_BK = 512

_DIMS = (((1,), (1,)), ((), ()))


def _mm3(ah, al, bh, bl):
    return jax.lax.dot_general(ah, bh, _DIMS, preferred_element_type=jnp.float32)


def _split(x):
    hi = x.astype(jnp.bfloat16)
    lo = (x - hi.astype(jnp.float32)).astype(jnp.bfloat16)
    return hi, lo


def _qk_kernel(x3, wq, wk, bq, bk, q_out, k_out):
    kk = pl.program_id(2)

    @pl.when(kk == 0)
    def _():
        q_out[...] = jnp.zeros_like(q_out)
        k_out[...] = jnp.zeros_like(k_out)

    xh, xl = _split(x3[0])
    wqh, wql = _split(wq[...])
    wkh, wkl = _split(wk[...])
    q_out[...] += _mm3(xh, xl, wqh, wql)
    k_out[...] += _mm3(xh, xl, wkh, wkl)

    @pl.when(kk == pl.num_programs(2) - 1)
    def _():
        q_out[...] += bq[...]
        k_out[...] += bk[...]


def _qk_call(clause_emb, wq, bq, wk, bk):
    grid = (_N // _BM, _H // _BN, _H // _BK)
    return pl.pallas_call(
        _qk_kernel,
        grid=grid,
        in_specs=[
            pl.BlockSpec((1, _BM, _BK), lambda i, j, k: (0, i, k)),
            pl.BlockSpec((_BN, _BK), lambda i, j, k: (j, k)),
            pl.BlockSpec((_BN, _BK), lambda i, j, k: (j, k)),
            pl.BlockSpec((1, _BN), lambda i, j, k: (0, j)),
            pl.BlockSpec((1, _BN), lambda i, j, k: (0, j)),
        ],
        out_specs=[
            pl.BlockSpec((_BM, _BN), lambda i, j, k: (i, j)),
            pl.BlockSpec((_BM, _BN), lambda i, j, k: (i, j)),
        ],
        out_shape=[
            jax.ShapeDtypeStruct((_N, _H), jnp.float32),
            jax.ShapeDtypeStruct((_N, _H), jnp.float32),
        ],
        compiler_params=pltpu.CompilerParams(
            dimension_semantics=("parallel", "parallel", "arbitrary"),
        ),
    )(clause_emb, wq, wk, bq.reshape(1, _H), bk.reshape(1, _H))


def _attn_kernel(q, k, a_out):
    kk = pl.program_id(2)

    @pl.when(kk == 0)
    def _():
        a_out[...] = jnp.zeros_like(a_out)

    qh, ql = _split(q[...])
    kh, kl = _split(k[...])
    a_out[...] += _mm3(qh, ql, kh, kl)

    @pl.when(kk == pl.num_programs(2) - 1)
    def _():
        a_out[...] *= _SCALE


def _attn_call(q, k):
    grid = (_N // _BM, _N // _BN, _H // _BK)
    return pl.pallas_call(
        _attn_kernel,
        grid=grid,
        in_specs=[
            pl.BlockSpec((_BM, _BK), lambda i, j, k: (i, k)),
            pl.BlockSpec((_BN, _BK), lambda i, j, k: (j, k)),
        ],
        out_specs=pl.BlockSpec((_BM, _BN), lambda i, j, k: (i, j)),
        out_shape=jax.ShapeDtypeStruct((_N, _N), jnp.float32),
        compiler_params=pltpu.CompilerParams(
            dimension_semantics=("parallel", "parallel", "arbitrary"),
        ),
    )(q, k)


_BR = 256
_RC = 1024
_RROWS = 2048
_NB = _RROWS // _BR
_NCHUNK = _TRI // _RC


def _tu(i):
    return (i * (2 * _N - 1 - i)) // 2


def _tl(p):
    return (p * (p - 1)) // 2


def _unrank(t0, tfun, hi0):
    def body(_, lh):
        lo, hi = lh
        mid = (lo + hi + 1) // 2
        pred = tfun(mid) <= t0
        return (jnp.where(pred, mid, lo), jnp.where(pred, hi, mid - 1))

    lo, _ = jax.lax.fori_loop(0, 11, body, (jnp.int32(0), jnp.int32(hi0)))
    return lo


def _load_unaligned(af, start):
    start_al = pl.multiple_of((start // 128) * 128, 128)
    sh = start % 128
    w = af[:, pl.ds(start_al, _RC + 128)]
    n = _RC + 128
    rolled = pltpu.roll(w, jax.lax.rem(n - sh, n), axis=1)
    return rolled[:, :_RC]


def _flatten_kernel(af, u_o, l_o):
    c = pl.program_id(0)
    t0 = c * _RC
    tvec = t0 + jax.lax.broadcasted_iota(jnp.int32, (1, _RC), 1)

    def u_body(state):
        i, acc = state
        T_i = _tu(i)
        v = _load_unaligned(af, t0 + (i * (_N + 1) + 1 - T_i))
        m = (tvec >= T_i) & (tvec < T_i + (_N - 1 - i))
        return (i + 1, jnp.where(m, v, acc))

    def u_cond(state):
        i, _ = state
        return (i <= _N - 2) & (_tu(i) < t0 + _RC)

    _, uacc = jax.lax.while_loop(
        u_cond, u_body,
        (_unrank(t0, _tu, _N - 2), jnp.zeros((1, _RC), jnp.float32)),
    )
    u_o[...] = uacc.reshape(1, 1, _RC)

    def l_body(state):
        p, acc = state
        T_p = _tl(p)
        v = _load_unaligned(af, t0 + (p * _N - T_p))
        m = (tvec >= T_p) & (tvec < T_p + p)
        return (p + 1, jnp.where(m, v, acc))

    def l_cond(state):
        p, _ = state
        return (p <= _N - 1) & (_tl(p) < t0 + _RC)

    _, lacc = jax.lax.while_loop(
        l_cond, l_body,
        (_unrank(t0, _tl, _N - 1), jnp.zeros((1, _RC), jnp.float32)),
    )
    l_o[...] = lacc.reshape(1, 1, _RC)


_NW = 32
_TW = _TRI // _NW
_SG = 16


def _sc_unrank(t0, tfun, hi0):
    def body(_, lh):
        lo, hi = lh
        mid = (lo + hi + 1) // 2
        pred = tfun(mid) <= t0
        return (jnp.where(pred, mid, lo), jnp.where(pred, hi, mid - 1))

    lo, _ = lax.fori_loop(0, 11, body, (jnp.int32(0), jnp.int32(hi0)))
    return lo


def _sc_phase(a_hbm, stage, build, out_hbm, t0, t1, tfun, lenfun, colfun, rmax):
    lane = jnp.arange(16, dtype=jnp.int32)

    r_start = _sc_unrank(t0, tfun, rmax)
    r_end = jnp.minimum(_sc_unrank(t1 - 1, tfun, rmax) + 1, rmax + 1)
    b0 = (r_start // 8) * 8
    n_groups = (r_end - b0 + _SG - 1) // _SG

    def group_body(g, _):
        b_raw = b0 + g * _SG
        b = pl.multiple_of(jnp.minimum(b_raw, _N - _SG), 8)
        pltpu.sync_copy(a_hbm.at[pl.ds(b * _N, _SG * _N)],
                        stage.at[pl.ds(0, _SG * _N)])
        row_lo = jnp.maximum(r_start, b_raw)
        row_hi = jnp.minimum(r_end, b_raw + _SG)

        def row_body(rr, _):
            T_r = tfun(rr)
            s = jnp.maximum(T_r, t0)
            e = jnp.minimum(T_r + lenfun(rr), t1)
            c0 = colfun(rr, s)
            d0 = s - t0
            nchunks = (jnp.maximum(e - s, 0) + 15) // 16
            src0 = (rr - b) * _N + c0

            def copy_body(kk, _):
                off = kk * 16
                build[pl.ds(d0 + off, 16)] = stage[pl.ds(src0 + off, 16)]
                return 0

            lax.fori_loop(0, nchunks, copy_body, 0, unroll=False)
            return 0

        lax.fori_loop(row_lo, row_hi, row_body, 0, unroll=False)
        return 0

    lax.fori_loop(0, n_groups, group_body, 0, unroll=False)
    pltpu.sync_copy(build.at[pl.ds(0, _TW)], out_hbm.at[pl.ds(t0, _TW)])


def _sc_flatten_call(a):
    mesh = plsc.VectorSubcoreMesh(core_axis_name="c", subcore_axis_name="s")

    @functools.partial(
        pl.kernel,
        out_type=[
            jax.ShapeDtypeStruct((_TRI,), jnp.float32),
            jax.ShapeDtypeStruct((_TRI,), jnp.float32),
        ],
        mesh=mesh,
        scratch_types=[
            pltpu.VMEM((_SG * _N + 16,), jnp.float32),
            pltpu.VMEM((_TW + 16,), jnp.float32),
        ],
    )
    def k(a_hbm, u_hbm, l_hbm, stage, build):
        wid = lax.axis_index("s") * 2 + lax.axis_index("c")
        t0 = wid * _TW
        t1 = t0 + _TW
        _sc_phase(
            a_hbm, stage, build, u_hbm, t0, t1,
            _tu, lambda i: _N - 1 - i,
            lambda i, s: i + 1 + (s - _tu(i)), _N - 2,
        )
        _sc_phase(
            a_hbm, stage, build, l_hbm, t0, t1,
            _tl, lambda p: p,
            lambda p, s: s - _tl(p), _N - 1,
        )

    return k(a.reshape(_N * _N))


def _flatten_call(af):
    return pl.pallas_call(
        _flatten_kernel,
        grid=(_NCHUNK,),
        in_specs=[pl.BlockSpec(memory_space=pltpu.VMEM)],
        out_specs=[
            pl.BlockSpec((1, 1, _RC), lambda c: (c, 0, 0)),
            pl.BlockSpec((1, 1, _RC), lambda c: (c, 0, 0)),
        ],
        out_shape=[
            jax.ShapeDtypeStruct((_NCHUNK, 1, _RC), jnp.float32),
            jax.ShapeDtypeStruct((_NCHUNK, 1, _RC), jnp.float32),
        ],
        compiler_params=pltpu.CompilerParams(
            dimension_semantics=("arbitrary",),
            vmem_limit_bytes=100 * 1024 * 1024,
        ),
    )(af)


def _reduce_kernel(u, l, logp_o, r_o, c_o, m_s, s_s, th_s, vals_s, t_s):
    b = pl.program_id(0)
    lane = jax.lax.broadcasted_iota(jnp.int32, (1, 128), 1)

    @pl.when(b == 0)
    def _():
        m_s[0, 0] = _NINF
        s_s[0, 0] = 0.0
        th_s[0, 0] = _NINF
        vals_s[...] = jnp.full((1, 128), _NINF, jnp.float32)
        t_s[...] = jnp.full((1, 128), _BIG, jnp.int32)

    rows = jax.lax.broadcasted_iota(jnp.int32, (_BR, _RC), 0)
    cols = jax.lax.broadcasted_iota(jnp.int32, (_BR, _RC), 1)
    tg = (b * _BR + rows) * _RC + cols
    v = jnp.where(tg < _TRI, u[...] + l[...], _NINF)

    bm = jnp.max(v)
    mo = m_s[0, 0]
    mn = jnp.maximum(mo, bm)
    s_s[0, 0] = s_s[0, 0] * jnp.exp(mo - mn) + jnp.sum(jnp.exp(v - mn))
    m_s[0, 0] = mn

    @pl.when(bm > th_s[0, 0])
    def _():
        cand_v = jnp.full((1, 128), _NINF, jnp.float32)
        cand_t = jnp.full((1, 128), _BIG, jnp.int32)
        vv = v
        for it in range(_TOPK):
            mv = jnp.max(vv)
            ts = jnp.min(jnp.where(vv == mv, tg, _BIG))
            cand_v = jnp.where(lane == _TOPK + it, mv, cand_v)
            cand_t = jnp.where(lane == _TOPK + it, ts, cand_t)
            vv = jnp.where(tg == ts, _NINF, vv)

        allv = jnp.where(lane < _TOPK, vals_s[...], cand_v)
        allt = jnp.where(lane < _TOPK, t_s[...], cand_t)
        nv = jnp.full((1, 128), _NINF, jnp.float32)
        nt = jnp.full((1, 128), _BIG, jnp.int32)
        for slot in range(_TOPK):
            mv = jnp.max(allv)
            ts = jnp.min(jnp.where(allv == mv, allt, _BIG))
            nv = jnp.where(lane == slot, mv, nv)
            nt = jnp.where(lane == slot, ts, nt)
            allv = jnp.where(allt == ts, _NINF, allv)
        vals_s[...] = nv
        t_s[...] = nt
        th_s[0, 0] = jnp.min(jnp.where(lane < _TOPK, nv, jnp.inf))

    @pl.when(b == _NB - 1)
    def _():
        lse = m_s[0, 0] + jnp.log(s_s[0, 0])
        t = t_s[...]
        lo = jnp.zeros((1, 128), jnp.int32)
        hi = jnp.full((1, 128), _N - 2, jnp.int32)
        for _it in range(11):
            mid = (lo + hi + 1) // 2
            tu = (mid * (2 * _N - 1 - mid)) // 2
            pred = tu <= t
            lo = jnp.where(pred, mid, lo)
            hi = jnp.where(pred, hi, mid - 1)
        tu_lo = (lo * (2 * _N - 1 - lo)) // 2
        logp_o[...] = vals_s[...] - lse
        r_o[...] = lo
        c_o[...] = t - tu_lo + lo + 1


def _reduce_call(u2, l2):
    return pl.pallas_call(
        _reduce_kernel,
        grid=(_NB,),
        in_specs=[
            pl.BlockSpec((_BR, _RC), lambda b: (b, 0)),
            pl.BlockSpec((_BR, _RC), lambda b: (b, 0)),
        ],
        out_specs=[
            pl.BlockSpec((1, 128), lambda b: (0, 0)),
            pl.BlockSpec((1, 128), lambda b: (0, 0)),
            pl.BlockSpec((1, 128), lambda b: (0, 0)),
        ],
        out_shape=[
            jax.ShapeDtypeStruct((1, 128), jnp.float32),
            jax.ShapeDtypeStruct((1, 128), jnp.int32),
            jax.ShapeDtypeStruct((1, 128), jnp.int32),
        ],
        scratch_shapes=[
            pltpu.SMEM((1, 1), jnp.float32),
            pltpu.SMEM((1, 1), jnp.float32),
            pltpu.SMEM((1, 1), jnp.float32),
            pltpu.VMEM((1, 128), jnp.float32),
            pltpu.VMEM((1, 128), jnp.int32),
        ],
        compiler_params=pltpu.CompilerParams(
            dimension_semantics=("arbitrary",),
        ),
    )(u2, l2)


def kernel(clause_emb, Wq_w, Wq_b, Wk_w, Wk_b, keep_mask):
    q, k = _qk_call(clause_emb, Wq_w, Wq_b, Wk_w, Wk_b)
    a = _attn_call(q, k)
    uf, lf = _sc_flatten_call(a)
    logp, rr, cc = _reduce_call(
        uf.reshape(_NCHUNK, _RC), lf.reshape(_NCHUNK, _RC)
    )
    return (logp[0, :_TOPK], rr[0, :_TOPK], cc[0, :_TOPK])
